# baseline (speedup 1.0000x reference)
import jax
import jax.numpy as jnp
from jax import lax
from jax.experimental import pallas as pl
from jax.experimental.pallas import tpu as pltpu
from jax.experimental.pallas import tpu_sc as plsc

NC = 2
NS = 16
L = 16


def _round_up(x, m):
    return (x + m - 1) // m * m


def _make_phase_a(M, D, B):
    NW = NC * NS
    CHUNK = B // NW
    NVEC = B // L
    VPW = NVEC // NS
    RANGE = _round_up(-(-M // NS), L)

    def body(fs_hbm, val_hbm, idx_hbm, frame_hbm,
             ids_hbm, obs_hbm,
             idx_all, idx_chunk, masks, m16, wslab, src_v, rows_v,
             stamp_v, frame_v, wsh_w, wsh_m, sem, sem2, sem3):
        cid = lax.axis_index("c")
        sid = lax.axis_index("s")
        g = cid * NS + sid

        pltpu.sync_copy(idx_hbm, idx_all)
        pltpu.sync_copy(idx_hbm.at[pl.ds(g * CHUNK, CHUNK)], idx_chunk)
        pltpu.sync_copy(frame_hbm, frame_v)

        fvec = frame_v[...]

        @pl.loop(0, CHUNK // L)
        def _fill(i):
            stamp_v[pl.ds(i * L, L)] = fvec

        cp_ids = pltpu.async_copy(
            idx_chunk, ids_hbm.at[pl.ds(g * CHUNK, CHUNK)], sem3)
        cp_fs = pltpu.async_copy(stamp_v, fs_hbm.at[idx_chunk], sem3)

        r0 = sid * RANGE
        iota = lax.iota(jnp.int32, L)
        rotk = (iota + (L - 1)) & (L - 1)
        is_top = iota == L - 1

        @pl.loop(0, VPW)
        def _dedup(j):
            v = sid * VPW + j
            iv = idx_all[pl.ds(v * L, L)]
            cs = lax.sort((iv << 4) | iota)
            _, nxt = plsc.sort_key_val(rotk, cs)
            win = ((cs >> 4) != (nxt >> 4)) | is_top
            plsc.store_scatter(m16, [cs & (L - 1)],
                               jnp.where(win, 1, 0))
            masks[pl.ds(v * L, L)] = m16[...]

        mo = sid * VPW * L
        pltpu.sync_copy(masks.at[pl.ds(mo, VPW * L)],
                        wsh_m.at[pl.ds(mo, VPW * L)])
        plsc.subcore_barrier()
        pltpu.sync_copy(wsh_m, masks)

        @pl.loop(0, NVEC, unroll=8)
        def _scan(v):
            iv = idx_all[pl.ds(v * L, L)]
            mv = masks[pl.ds(v * L, L)]
            local = iv - r0
            m = (mv != 0) & (local >= 0) & (local < RANGE)
            safe = jnp.where(m, local, 0)
            plsc.store_scatter(wslab, [safe], v * L + iota, mask=m)

        pltpu.sync_copy(wslab, wsh_w.at[pl.ds(sid * RANGE, RANGE)])
        plsc.subcore_barrier()

        H = CHUNK // 2
        pltpu.async_copy(wsh_w.at[idx_chunk], src_v, sem).wait()
        cp_g0 = pltpu.async_copy(
            val_hbm.at[src_v.at[pl.ds(0, H)]], rows_v.at[pl.ds(0, H)], sem)
        cp_g1 = pltpu.async_copy(
            val_hbm.at[src_v.at[pl.ds(H, H)]], rows_v.at[pl.ds(H, H)], sem2)
        cp_g0.wait()
        cp_w0 = pltpu.async_copy(
            rows_v.at[pl.ds(0, H)], obs_hbm.at[pl.ds(g * CHUNK, H)], sem)
        cp_g1.wait()
        cp_w1 = pltpu.async_copy(
            rows_v.at[pl.ds(H, H)], obs_hbm.at[pl.ds(g * CHUNK + H, H)], sem2)
        cp_w0.wait()
        cp_w1.wait()

        cp_ids.wait()
        cp_fs.wait()

    return pl.kernel(
        body,
        out_type=(
            jax.ShapeDtypeStruct((B,), jnp.int32),
            jax.ShapeDtypeStruct((B, D), jnp.float32),
        ),
        mesh=plsc.VectorSubcoreMesh(
            core_axis_name="c", subcore_axis_name="s",
            num_cores=NC, num_subcores=NS,
        ),
        scratch_types=[
            pltpu.VMEM((B,), jnp.int32),
            pltpu.VMEM((CHUNK,), jnp.int32),
            pltpu.VMEM((B,), jnp.int32),
            pltpu.VMEM((L,), jnp.int32),
            pltpu.VMEM((RANGE,), jnp.int32),
            pltpu.VMEM((CHUNK,), jnp.int32),
            pltpu.VMEM((CHUNK, D), jnp.float32),
            pltpu.VMEM((CHUNK,), jnp.int32),
            pltpu.VMEM((L,), jnp.int32),
            pltpu.VMEM_SHARED((NS * RANGE,), jnp.int32),
            pltpu.VMEM_SHARED((B,), jnp.int32),
            pltpu.SemaphoreType.DMA,
            pltpu.SemaphoreType.DMA,
            pltpu.SemaphoreType.DMA,
        ],
        compiler_params=pltpu.CompilerParams(needs_layout_passes=False),
        name="tracklet_obs_ids_frame",
    )


def _make_phase_b(M, D, B):
    NW = NC * NS
    CHUNK = B // NW

    def body(mem_hbm, obs_hbm, idx_hbm,
             idx_chunk0, idx_chunk1, rows_v, sem, sem2):
        cid = lax.axis_index("c")
        sid = lax.axis_index("s")
        g = cid * NS + sid
        H = CHUNK // 2

        pltpu.sync_copy(idx_hbm.at[pl.ds(g * CHUNK, H)], idx_chunk0)
        cp_r0 = pltpu.async_copy(
            obs_hbm.at[pl.ds(g * CHUNK, H)], rows_v.at[pl.ds(0, H)], sem)
        pltpu.sync_copy(idx_hbm.at[pl.ds(g * CHUNK + H, H)], idx_chunk1)
        cp_r1 = pltpu.async_copy(
            obs_hbm.at[pl.ds(g * CHUNK + H, H)], rows_v.at[pl.ds(H, H)], sem2)
        cp_r0.wait()
        cp_s0 = pltpu.async_copy(
            rows_v.at[pl.ds(0, H)], mem_hbm.at[idx_chunk0], sem)
        cp_r1.wait()
        cp_s1 = pltpu.async_copy(
            rows_v.at[pl.ds(H, H)], mem_hbm.at[idx_chunk1], sem2)
        cp_s0.wait()
        cp_s1.wait()

    return pl.kernel(
        body,
        out_type=(),
        mesh=plsc.VectorSubcoreMesh(
            core_axis_name="c", subcore_axis_name="s",
            num_cores=NC, num_subcores=NS,
        ),
        scratch_types=[
            pltpu.VMEM((CHUNK // 2,), jnp.int32),
            pltpu.VMEM((CHUNK // 2,), jnp.int32),
            pltpu.VMEM((CHUNK, D), jnp.float32),
            pltpu.SemaphoreType.DMA,
            pltpu.SemaphoreType.DMA,
        ],
        compiler_params=pltpu.CompilerParams(needs_layout_passes=False),
        name="tracklet_row_scatter",
    )


def _make_tc_copy(M, D, blocks):
    BM = M // blocks

    def body(src_ref, dst_ref):
        dst_ref[...] = src_ref[...]

    return pl.pallas_call(
        body,
        grid=(blocks,),
        in_specs=[pl.BlockSpec((BM, D), lambda i: (i, 0))],
        out_specs=pl.BlockSpec((BM, D), lambda i: (i, 0)),
        out_shape=jax.ShapeDtypeStruct((M, D), jnp.float32),
        name="tracklet_mem_copy",
    )


def kernel(mem, val, frame_state, idx, frame):
    M, D = mem.shape
    B = idx.shape[0]
    frame_arr = jnp.full((L,), frame, dtype=jnp.int32)
    fs_ref = jax.new_ref(frame_state)
    ids, obs = _make_phase_a(M, D, B)(fs_ref, val, idx, frame_arr)
    mem_ref = jax.new_ref(_make_tc_copy(M, D, 25)(mem))
    _make_phase_b(M, D, B)(mem_ref, obs, idx)
    return mem_ref[...], fs_ref[...], ids, obs


# speedup vs baseline: 1.0408x; 1.0408x over previous
import jax
import jax.numpy as jnp
from jax import lax
from jax.experimental import pallas as pl
from jax.experimental.pallas import tpu as pltpu
from jax.experimental.pallas import tpu_sc as plsc

NC = 2
NS = 16
L = 16


def _round_up(x, m):
    return (x + m - 1) // m * m


def _make_phase_a(M, D, B):
    NW = NC * NS
    CHUNK = B // NW
    NVEC = B // L
    VPW = NVEC // NS
    RANGE = _round_up(-(-M // NS), L)

    SENT = 1 << 30

    def body(fs_hbm, val_hbm, idx_hbm, frame_hbm,
             ids_hbm, obs_hbm,
             ded_idx, idx_chunk, midx_all, m16, wslab, src_v, rows_v,
             stamp_v, frame_v, wsh_w, wsh_m, sem, sem2, sem3):
        cid = lax.axis_index("c")
        sid = lax.axis_index("s")
        g = cid * NS + sid

        mo = sid * VPW * L
        pltpu.sync_copy(idx_hbm.at[pl.ds(mo, VPW * L)], ded_idx)
        pltpu.sync_copy(idx_hbm.at[pl.ds(g * CHUNK, CHUNK)], idx_chunk)
        pltpu.sync_copy(frame_hbm, frame_v)

        fvec = frame_v[...]

        @pl.loop(0, CHUNK // L)
        def _fill(i):
            stamp_v[pl.ds(i * L, L)] = fvec

        cp_ids = pltpu.async_copy(
            idx_chunk, ids_hbm.at[pl.ds(g * CHUNK, CHUNK)], sem3)
        cp_fs = pltpu.async_copy(stamp_v, fs_hbm.at[idx_chunk], sem3)

        r0 = sid * RANGE
        iota = lax.iota(jnp.int32, L)
        rotk = (iota + (L - 1)) & (L - 1)
        is_top = iota == L - 1

        @pl.loop(0, VPW)
        def _dedup(j):
            iv = ded_idx[pl.ds(j * L, L)]
            cs = lax.sort((iv << 4) | iota)
            _, nxt = plsc.sort_key_val(rotk, cs)
            win = ((cs >> 4) != (nxt >> 4)) | is_top
            plsc.store_scatter(m16, [cs & (L - 1)],
                               jnp.where(win, 1, 0))
            midx_all[pl.ds(mo + j * L, L)] = jnp.where(
                m16[...] != 0, iv, SENT)

        pltpu.sync_copy(midx_all.at[pl.ds(mo, VPW * L)],
                        wsh_m.at[pl.ds(mo, VPW * L)])
        plsc.subcore_barrier()
        pltpu.sync_copy(wsh_m, midx_all)

        @pl.loop(0, NVEC, unroll=8)
        def _scan(v):
            local = midx_all[pl.ds(v * L, L)] - r0
            m = (local >= 0) & (local < RANGE)
            safe = jnp.where(m, local, 0)
            plsc.store_scatter(wslab, [safe], v * L + iota, mask=m)

        pltpu.sync_copy(wslab, wsh_w.at[pl.ds(sid * RANGE, RANGE)])
        plsc.subcore_barrier()

        H = CHUNK // 2
        pltpu.async_copy(wsh_w.at[idx_chunk], src_v, sem).wait()
        cp_g0 = pltpu.async_copy(
            val_hbm.at[src_v.at[pl.ds(0, H)]], rows_v.at[pl.ds(0, H)], sem)
        cp_g1 = pltpu.async_copy(
            val_hbm.at[src_v.at[pl.ds(H, H)]], rows_v.at[pl.ds(H, H)], sem2)
        cp_g0.wait()
        cp_w0 = pltpu.async_copy(
            rows_v.at[pl.ds(0, H)], obs_hbm.at[pl.ds(g * CHUNK, H)], sem)
        cp_g1.wait()
        cp_w1 = pltpu.async_copy(
            rows_v.at[pl.ds(H, H)], obs_hbm.at[pl.ds(g * CHUNK + H, H)], sem2)
        cp_w0.wait()
        cp_w1.wait()

        cp_ids.wait()
        cp_fs.wait()

    return pl.kernel(
        body,
        out_type=(
            jax.ShapeDtypeStruct((B,), jnp.int32),
            jax.ShapeDtypeStruct((B, D), jnp.float32),
        ),
        mesh=plsc.VectorSubcoreMesh(
            core_axis_name="c", subcore_axis_name="s",
            num_cores=NC, num_subcores=NS,
        ),
        scratch_types=[
            pltpu.VMEM((B // NS,), jnp.int32),
            pltpu.VMEM((CHUNK,), jnp.int32),
            pltpu.VMEM((B,), jnp.int32),
            pltpu.VMEM((L,), jnp.int32),
            pltpu.VMEM((RANGE,), jnp.int32),
            pltpu.VMEM((CHUNK,), jnp.int32),
            pltpu.VMEM((CHUNK, D), jnp.float32),
            pltpu.VMEM((CHUNK,), jnp.int32),
            pltpu.VMEM((L,), jnp.int32),
            pltpu.VMEM_SHARED((NS * RANGE,), jnp.int32),
            pltpu.VMEM_SHARED((B,), jnp.int32),
            pltpu.SemaphoreType.DMA,
            pltpu.SemaphoreType.DMA,
            pltpu.SemaphoreType.DMA,
        ],
        compiler_params=pltpu.CompilerParams(needs_layout_passes=False),
        name="tracklet_obs_ids_frame",
    )


def _make_phase_b(M, D, B):
    NW = NC * NS
    CHUNK = B // NW

    def body(mem_hbm, obs_hbm, idx_hbm,
             idx_chunk0, idx_chunk1, rows_v, sem, sem2):
        cid = lax.axis_index("c")
        sid = lax.axis_index("s")
        g = cid * NS + sid
        H = CHUNK // 2

        pltpu.sync_copy(idx_hbm.at[pl.ds(g * CHUNK, H)], idx_chunk0)
        cp_r0 = pltpu.async_copy(
            obs_hbm.at[pl.ds(g * CHUNK, H)], rows_v.at[pl.ds(0, H)], sem)
        pltpu.sync_copy(idx_hbm.at[pl.ds(g * CHUNK + H, H)], idx_chunk1)
        cp_r1 = pltpu.async_copy(
            obs_hbm.at[pl.ds(g * CHUNK + H, H)], rows_v.at[pl.ds(H, H)], sem2)
        cp_r0.wait()
        cp_s0 = pltpu.async_copy(
            rows_v.at[pl.ds(0, H)], mem_hbm.at[idx_chunk0], sem)
        cp_r1.wait()
        cp_s1 = pltpu.async_copy(
            rows_v.at[pl.ds(H, H)], mem_hbm.at[idx_chunk1], sem2)
        cp_s0.wait()
        cp_s1.wait()

    return pl.kernel(
        body,
        out_type=(),
        mesh=plsc.VectorSubcoreMesh(
            core_axis_name="c", subcore_axis_name="s",
            num_cores=NC, num_subcores=NS,
        ),
        scratch_types=[
            pltpu.VMEM((CHUNK // 2,), jnp.int32),
            pltpu.VMEM((CHUNK // 2,), jnp.int32),
            pltpu.VMEM((CHUNK, D), jnp.float32),
            pltpu.SemaphoreType.DMA,
            pltpu.SemaphoreType.DMA,
        ],
        compiler_params=pltpu.CompilerParams(needs_layout_passes=False),
        name="tracklet_row_scatter",
    )


def _make_tc_copy(M, D, blocks):
    BM = M // blocks

    def body(src_ref, dst_ref):
        dst_ref[...] = src_ref[...]

    return pl.pallas_call(
        body,
        grid=(blocks,),
        in_specs=[pl.BlockSpec((BM, D), lambda i: (i, 0))],
        out_specs=pl.BlockSpec((BM, D), lambda i: (i, 0)),
        out_shape=jax.ShapeDtypeStruct((M, D), jnp.float32),
        name="tracklet_mem_copy",
    )


def kernel(mem, val, frame_state, idx, frame):
    M, D = mem.shape
    B = idx.shape[0]
    frame_arr = jnp.full((L,), frame, dtype=jnp.int32)
    fs_ref = jax.new_ref(frame_state)
    ids, obs = _make_phase_a(M, D, B)(fs_ref, val, idx, frame_arr)
    mem_ref = jax.new_ref(_make_tc_copy(M, D, 25)(mem))
    _make_phase_b(M, D, B)(mem_ref, obs, idx)
    return mem_ref[...], fs_ref[...], ids, obs
